# baseline (speedup 1.0000x reference)
import functools

import jax
import jax.numpy as jnp
from jax import lax
from jax.experimental import pallas as pl
from jax.experimental.pallas import tpu as pltpu
from jax.experimental.pallas import tpu_sc as plsc

N = 10000
D = 128
H = 64
E = 320000
NEG = 50000

NC = 2
NS = 16
NW = NC * NS

C = 128
N_PAD = 10240
RPS = N_PAD // NS
DUMMY_ROW = N

NB = 4
NBS = 2


def _round8(x):
    return -(-x // 8) * 8


NCH = _round8(-(-E // (NW * C)))
NCHX = NCH + 8
E_PAD = NCH * NW * C
EPW = NCH * C

EA = E + NEG
NCHA = _round8(-(-EA // (NW * C)))
NCHAX = NCHA + 8
EA_PAD = NCHA * NW * C
EPWA = NCHA * C
EA_PADX = NCHAX * NW * C
EPWAX = NCHAX * C

BR = 1000
BK = 2048
NBK = EA_PADX // BK


def _mesh():
    return plsc.VectorSubcoreMesh(
        core_axis_name="c", subcore_axis_name="s", num_cores=NC, num_subcores=NS
    )


_SC_PARAMS = pltpu.CompilerParams(use_tc_tiling_on_sc=False)


@functools.partial(
    pl.kernel,
    out_type=jax.ShapeDtypeStruct((NC, N_PAD, 8), jnp.float32),
    mesh=_mesh(),
    compiler_params=_SC_PARAMS,
    scratch_types=[
        pltpu.VMEM_SHARED((N_PAD, 8), jnp.float32),
        pltpu.VMEM((NCHX, C), jnp.int32),
        pltpu.VMEM((C, 8), jnp.float32),
    ],
)
def _deg_kernel(dst_hbm, zeros_hbm, ones_hbm, out_hbm, acc_sh, dst_v, ones_v):
    c = lax.axis_index("c")
    s = lax.axis_index("s")
    w = c * NS + s
    pltpu.sync_copy(zeros_hbm.at[pl.ds(s * RPS, RPS)], acc_sh.at[pl.ds(s * RPS, RPS)])
    pltpu.sync_copy(ones_hbm, ones_v)
    pltpu.sync_copy(dst_hbm.at[pl.ds(w * NCHX, NCHX)], dst_v)
    plsc.subcore_barrier()

    def body(j, carry):
        pltpu.sync_copy(ones_v, acc_sh.at[dst_v.at[j]], add=True)
        return carry

    lax.fori_loop(0, NCHX, body, 0)
    plsc.subcore_barrier()
    pltpu.sync_copy(acc_sh.at[pl.ds(s * RPS, RPS)], out_hbm.at[c].at[pl.ds(s * RPS, RPS)])


@functools.partial(
    pl.kernel,
    out_type=jax.ShapeDtypeStruct((NC, N_PAD, H), jnp.float32),
    mesh=_mesh(),
    compiler_params=_SC_PARAMS,
    scratch_types=[
        pltpu.VMEM_SHARED((N_PAD, H), jnp.float32),
        pltpu.VMEM((NCHX, C), jnp.int32),
        pltpu.VMEM((NCHX, C), jnp.int32),
    ]
    + [pltpu.VMEM((C, H), jnp.float32) for _ in range(NB)]
    + [pltpu.SemaphoreType.DMA for _ in range(NB)],
)
def _prop_kernel(m_hbm, src_hbm, dst_hbm, zeros_hbm, out_hbm,
                 acc_sh, src_v, dst_v, *rows_and_sems):
    rows = rows_and_sems[:NB]
    sems = rows_and_sems[NB:]
    c = lax.axis_index("c")
    s = lax.axis_index("s")
    w = c * NS + s
    pltpu.sync_copy(zeros_hbm.at[pl.ds(s * RPS, RPS)], acc_sh.at[pl.ds(s * RPS, RPS)])
    pltpu.sync_copy(src_hbm.at[pl.ds(w * NCHX, NCHX)], src_v)
    pltpu.sync_copy(dst_hbm.at[pl.ds(w * NCHX, NCHX)], dst_v)
    plsc.subcore_barrier()

    for b in range(NB):
        pltpu.async_copy(m_hbm.at[src_v.at[b]], rows[b], sems[b])

    def body(j0, carry):
        for b in range(NB):
            j = j0 * NB + b
            pltpu.make_async_copy(m_hbm.at[src_v.at[j]], rows[b], sems[b]).wait()
            pltpu.sync_copy(rows[b], acc_sh.at[dst_v.at[j]], add=True)
            pltpu.async_copy(m_hbm.at[src_v.at[j + NB]], rows[b], sems[b])
        return carry

    lax.fori_loop(0, NCH // NB, body, 0)
    for b in range(NB):
        pltpu.make_async_copy(m_hbm.at[src_v.at[b]], rows[b], sems[b]).wait()

    plsc.subcore_barrier()
    pltpu.sync_copy(acc_sh.at[pl.ds(s * RPS, RPS)], out_hbm.at[c].at[pl.ds(s * RPS, RPS)])


@functools.partial(
    pl.kernel,
    out_type=jax.ShapeDtypeStruct((EA_PADX, H), jnp.float32),
    mesh=_mesh(),
    compiler_params=_SC_PARAMS,
    scratch_types=[
        pltpu.VMEM((NCHAX, C), jnp.int32),
        pltpu.VMEM((NCHAX, C), jnp.int32),
        pltpu.VMEM((C, H), jnp.float32),
    ]
    + [pltpu.VMEM((C, H), jnp.float32) for _ in range(2 * NBS)]
    + [pltpu.SemaphoreType.DMA for _ in range(2 * NBS)],
)
def _edgeprod_kernel(reps_hbm, src_hbm, dst_hbm, p_out,
                     src_v, dst_v, prod_v, *bufs_and_sems):
    rs = bufs_and_sems[0:NBS]
    rd = bufs_and_sems[NBS:2 * NBS]
    sems_s = bufs_and_sems[2 * NBS:3 * NBS]
    sems_d = bufs_and_sems[3 * NBS:4 * NBS]
    c = lax.axis_index("c")
    s = lax.axis_index("s")
    w = c * NS + s
    pltpu.sync_copy(src_hbm.at[pl.ds(w * NCHAX, NCHAX)], src_v)
    pltpu.sync_copy(dst_hbm.at[pl.ds(w * NCHAX, NCHAX)], dst_v)

    for b in range(NBS):
        pltpu.async_copy(reps_hbm.at[src_v.at[b]], rs[b], sems_s[b])
        pltpu.async_copy(reps_hbm.at[dst_v.at[b]], rd[b], sems_d[b])

    def body(j0, carry):
        for b in range(NBS):
            j = j0 * NBS + b
            pltpu.make_async_copy(reps_hbm.at[src_v.at[j]], rs[b], sems_s[b]).wait()
            pltpu.make_async_copy(reps_hbm.at[dst_v.at[j]], rd[b], sems_d[b]).wait()

            def mul_body(i0, carry2):
                for ee in range(4):
                    for k in range(H // 16):
                        e = i0 * 4 + ee
                        sl = pl.ds(k * 16, 16)
                        prod_v[e, sl] = rs[b][e, sl] * rd[b][e, sl]
                return carry2

            lax.fori_loop(0, C // 4, mul_body, 0)
            pltpu.async_copy(reps_hbm.at[src_v.at[j + NBS]], rs[b], sems_s[b])
            pltpu.async_copy(reps_hbm.at[dst_v.at[j + NBS]], rd[b], sems_d[b])
            pltpu.sync_copy(prod_v, p_out.at[pl.ds(w * EPWAX + j * C, C)])
        return carry

    lax.fori_loop(0, NCHA // NBS, body, 0)
    for b in range(NBS):
        pltpu.make_async_copy(reps_hbm.at[src_v.at[b]], rs[b], sems_s[b]).wait()
        pltpu.make_async_copy(reps_hbm.at[dst_v.at[b]], rd[b], sems_d[b]).wait()


def _tc1_body(f_ref, w_ref, d0_ref, d1_ref, m1p_ref, dinv_ref):
    deg = d0_ref[:, :1] + d1_ref[:, :1] + 1.0
    dinv = lax.rsqrt(deg)
    m1 = jnp.dot(f_ref[...], w_ref[...], preferred_element_type=jnp.float32)
    m1p_ref[...] = dinv * m1
    dinv_ref[...] = jnp.broadcast_to(dinv, (BR, 8))


def _tc1(features, W1, deg0, deg1):
    return pl.pallas_call(
        _tc1_body,
        grid=(N // BR,),
        in_specs=[
            pl.BlockSpec((BR, D), lambda i: (i, 0)),
            pl.BlockSpec((D, H), lambda i: (0, 0)),
            pl.BlockSpec((BR, 8), lambda i: (i, 0)),
            pl.BlockSpec((BR, 8), lambda i: (i, 0)),
        ],
        out_specs=[
            pl.BlockSpec((BR, H), lambda i: (i, 0)),
            pl.BlockSpec((BR, 8), lambda i: (i, 0)),
        ],
        out_shape=[
            jax.ShapeDtypeStruct((N, H), jnp.float32),
            jax.ShapeDtypeStruct((N, 8), jnp.float32),
        ],
    )(features, W1, deg0, deg1)


def _tc2_body(a0_ref, a1_ref, m1p_ref, dinv_ref, b1_ref, w2_ref, m2p_ref):
    dinv = dinv_ref[:, :1]
    pre = dinv * (a0_ref[...] + a1_ref[...] + m1p_ref[...]) + b1_ref[...]
    h = jnp.maximum(pre, 0.0)
    m2 = jnp.dot(h, w2_ref[...], preferred_element_type=jnp.float32)
    m2p_ref[...] = dinv * m2


def _tc2(acc0, acc1, m1p, dinv, b1, W2):
    return pl.pallas_call(
        _tc2_body,
        grid=(N // BR,),
        in_specs=[
            pl.BlockSpec((BR, H), lambda i: (i, 0)),
            pl.BlockSpec((BR, H), lambda i: (i, 0)),
            pl.BlockSpec((BR, H), lambda i: (i, 0)),
            pl.BlockSpec((BR, 8), lambda i: (i, 0)),
            pl.BlockSpec((1, H), lambda i: (0, 0)),
            pl.BlockSpec((H, H), lambda i: (0, 0)),
        ],
        out_specs=pl.BlockSpec((BR, H), lambda i: (i, 0)),
        out_shape=jax.ShapeDtypeStruct((N, H), jnp.float32),
    )(acc0, acc1, m1p, dinv, b1, W2)


def _tc3_body(a0_ref, a1_ref, m2p_ref, dinv_ref, b2_ref, reps_ref):
    dinv = dinv_ref[:, :1]
    reps_ref[...] = dinv * (a0_ref[...] + a1_ref[...] + m2p_ref[...]) + b2_ref[...]


def _tc3(acc0, acc1, m2p, dinv, b2):
    return pl.pallas_call(
        _tc3_body,
        grid=(N // BR,),
        in_specs=[
            pl.BlockSpec((BR, H), lambda i: (i, 0)),
            pl.BlockSpec((BR, H), lambda i: (i, 0)),
            pl.BlockSpec((BR, H), lambda i: (i, 0)),
            pl.BlockSpec((BR, 8), lambda i: (i, 0)),
            pl.BlockSpec((1, H), lambda i: (0, 0)),
        ],
        out_specs=pl.BlockSpec((BR, H), lambda i: (i, 0)),
        out_shape=jax.ShapeDtypeStruct((N, H), jnp.float32),
    )(acc0, acc1, m2p, dinv, b2)


def _score_body(p_ref, src_ref, dst_ref, t_ref, num_ref, den_ref, acc_ref):
    i = pl.program_id(0)

    @pl.when(i == 0)
    def _():
        acc_ref[0] = 0.0
        acc_ref[1] = 0.0

    p = jnp.sum(p_ref[...], axis=1, keepdims=True)
    m = src_ref[...] < dst_ref[...]
    mf = m.astype(jnp.float32)
    acc_ref[0] += jnp.sum(jnp.where(m, (p - t_ref[...]) ** 2, 0.0))
    acc_ref[1] += jnp.sum(mf)

    @pl.when(i == NBK - 1)
    def _():
        num_ref[0, 0] = acc_ref[0]
        den_ref[0, 0] = acc_ref[1]


def _tc_score(P, srcA, dstA, tgt):
    return pl.pallas_call(
        _score_body,
        grid=(NBK,),
        in_specs=[
            pl.BlockSpec((BK, H), lambda i: (i, 0)),
            pl.BlockSpec((BK, 1), lambda i: (i, 0)),
            pl.BlockSpec((BK, 1), lambda i: (i, 0)),
            pl.BlockSpec((BK, 1), lambda i: (i, 0)),
        ],
        out_specs=[
            pl.BlockSpec(memory_space=pltpu.SMEM),
            pl.BlockSpec(memory_space=pltpu.SMEM),
        ],
        out_shape=[
            jax.ShapeDtypeStruct((1, 1), jnp.float32),
            jax.ShapeDtypeStruct((1, 1), jnp.float32),
        ],
        scratch_shapes=[pltpu.SMEM((2,), jnp.float32)],
    )(P, srcA, dstA, tgt)


def _chunked(x, n_real, nch, nchx, fill):
    pad = NW * nch * C - n_real
    xp = jnp.concatenate([x, jnp.full((pad,), fill, x.dtype)])
    xr = xp.reshape(NW, nch, C)
    dummy = jnp.full((NW, nchx - nch, C), fill, x.dtype)
    return jnp.concatenate([xr, dummy], axis=1).reshape(NW * nchx, C)


def kernel(features, edge_index, neg_edges, W1, b1, W2, b2):
    src = edge_index[0]
    dst = edge_index[1]

    src_p = _chunked(src, E, NCH, NCHX, 0)
    dst_p = _chunked(dst, E, NCH, NCHX, DUMMY_ROW)

    srcA = jnp.concatenate([src, neg_edges[0]])
    dstA = jnp.concatenate([dst, neg_edges[1]])
    tgtA = jnp.concatenate([jnp.ones((E,), jnp.float32),
                            jnp.zeros((NEG,), jnp.float32)])
    srcA_p = _chunked(srcA, EA, NCHA, NCHAX, 0)
    dstA_p = _chunked(dstA, EA, NCHA, NCHAX, 0)
    tgt_p = _chunked(tgtA, EA, NCHA, NCHAX, 0.0)

    zeros8 = jnp.zeros((N_PAD, 8), jnp.float32)
    zerosH = jnp.zeros((N_PAD, H), jnp.float32)
    ones8 = jnp.ones((C, 8), jnp.float32)

    deg_pair = _deg_kernel(dst_p, zeros8, ones8)
    deg0 = deg_pair[0, :N, :]
    deg1 = deg_pair[1, :N, :]

    m1p, dinv = _tc1(features, W1, deg0, deg1)

    acc1 = _prop_kernel(m1p, src_p, dst_p, zerosH)

    m2p = _tc2(acc1[0, :N, :], acc1[1, :N, :], m1p, dinv, b1.reshape(1, H), W2)

    acc2 = _prop_kernel(m2p, src_p, dst_p, zerosH)

    reps = _tc3(acc2[0, :N, :], acc2[1, :N, :], m2p, dinv, b2.reshape(1, H))

    P = _edgeprod_kernel(reps, srcA_p, dstA_p)

    num, den = _tc_score(P, srcA_p.reshape(EA_PADX, 1), dstA_p.reshape(EA_PADX, 1),
                         tgt_p.reshape(EA_PADX, 1))
    rec_loss = (num[0, 0] * jnp.float32(N)) / den[0, 0]
    return reps, rec_loss


# speedup vs baseline: 1.9542x; 1.9542x over previous
import functools

import jax
import jax.numpy as jnp
from jax import lax
from jax.experimental import pallas as pl
from jax.experimental.pallas import tpu as pltpu
from jax.experimental.pallas import tpu_sc as plsc

N = 10000
D = 128
H = 64
E = 320000
NEG = 50000

NC = 2
NS = 16
NW = NC * NS

CC = 512
N_PAD = 10240
RPS = N_PAD // NS
DUMMY_ROW = N

G = -(-E // (NW * CC))
E_PAD = G * NW * CC
EPW = G * CC

EA = E + NEG
GA = -(-EA // (NW * CC))
EA_PAD = GA * NW * CC
EPWA = GA * CC

BR = 1000
BK = 2048
NBK = EA_PAD // BK


def _mesh():
    return plsc.VectorSubcoreMesh(
        core_axis_name="c", subcore_axis_name="s", num_cores=NC, num_subcores=NS
    )


_SC_PARAMS = pltpu.CompilerParams(use_tc_tiling_on_sc=False)


@functools.partial(
    pl.kernel,
    out_type=jax.ShapeDtypeStruct((NC, N_PAD, 8), jnp.float32),
    mesh=_mesh(),
    compiler_params=_SC_PARAMS,
    scratch_types=[
        pltpu.VMEM_SHARED((N_PAD, 8), jnp.float32),
        pltpu.VMEM((G, CC), jnp.int32),
        pltpu.VMEM((CC, 8), jnp.float32),
    ],
)
def _deg_kernel(dst_hbm, zeros_hbm, ones_hbm, out_hbm, acc_sh, dst_v, ones_v):
    c = lax.axis_index("c")
    s = lax.axis_index("s")
    w = c * NS + s
    pltpu.sync_copy(zeros_hbm.at[pl.ds(s * RPS, RPS)], acc_sh.at[pl.ds(s * RPS, RPS)])
    pltpu.sync_copy(ones_hbm, ones_v)
    pltpu.sync_copy(dst_hbm.at[w], dst_v)
    plsc.subcore_barrier()

    def body(j, carry):
        pltpu.sync_copy(ones_v, acc_sh.at[dst_v.at[j]], add=True)
        return carry

    lax.fori_loop(0, G, body, 0)
    plsc.subcore_barrier()
    pltpu.sync_copy(acc_sh.at[pl.ds(s * RPS, RPS)], out_hbm.at[c].at[pl.ds(s * RPS, RPS)])


@functools.partial(
    pl.kernel,
    out_type=jax.ShapeDtypeStruct((NC, N_PAD, H), jnp.float32),
    mesh=_mesh(),
    compiler_params=_SC_PARAMS,
    scratch_types=[
        pltpu.VMEM_SHARED((N_PAD, H), jnp.float32),
        pltpu.VMEM((G, CC), jnp.int32),
        pltpu.VMEM((G, CC), jnp.int32),
        pltpu.VMEM((CC, H), jnp.float32),
        pltpu.SemaphoreType.DMA,
    ],
)
def _prop_kernel(m_hbm, src_hbm, dst_hbm, zeros_hbm, out_hbm,
                 acc_sh, src_v, dst_v, rows_v, sem):
    c = lax.axis_index("c")
    s = lax.axis_index("s")
    w = c * NS + s
    pltpu.sync_copy(zeros_hbm.at[pl.ds(s * RPS, RPS)], acc_sh.at[pl.ds(s * RPS, RPS)])
    pltpu.sync_copy(src_hbm.at[w], src_v)
    pltpu.sync_copy(dst_hbm.at[w], dst_v)
    plsc.subcore_barrier()

    def body(j, carry):
        pltpu.async_copy(m_hbm.at[src_v.at[j]], rows_v, sem).wait()
        pltpu.sync_copy(rows_v, acc_sh.at[dst_v.at[j]], add=True)
        return carry

    lax.fori_loop(0, G, body, 0)
    plsc.subcore_barrier()
    pltpu.sync_copy(acc_sh.at[pl.ds(s * RPS, RPS)], out_hbm.at[c].at[pl.ds(s * RPS, RPS)])


@functools.partial(
    pl.kernel,
    out_type=jax.ShapeDtypeStruct((EA_PAD, H), jnp.float32),
    mesh=_mesh(),
    compiler_params=_SC_PARAMS,
    scratch_types=[
        pltpu.VMEM((GA, CC), jnp.int32),
        pltpu.VMEM((GA, CC), jnp.int32),
        pltpu.VMEM((CC, H), jnp.float32),
        pltpu.VMEM((CC, H), jnp.float32),
        pltpu.SemaphoreType.DMA,
        pltpu.SemaphoreType.DMA,
    ],
)
def _edgeprod_kernel(reps_hbm, src_hbm, dst_hbm, p_out,
                     src_v, dst_v, rs_v, rd_v, sem_s, sem_d):
    c = lax.axis_index("c")
    s = lax.axis_index("s")
    w = c * NS + s
    pltpu.sync_copy(src_hbm.at[w], src_v)
    pltpu.sync_copy(dst_hbm.at[w], dst_v)

    def body(j, carry):
        cs = pltpu.async_copy(reps_hbm.at[src_v.at[j]], rs_v, sem_s)
        cd = pltpu.async_copy(reps_hbm.at[dst_v.at[j]], rd_v, sem_d)
        cs.wait()
        cd.wait()

        def mul_body(i0, carry2):
            for ee in range(4):
                for k in range(H // 16):
                    e = i0 * 4 + ee
                    sl = pl.ds(k * 16, 16)
                    rs_v[e, sl] = rs_v[e, sl] * rd_v[e, sl]
            return carry2

        lax.fori_loop(0, CC // 4, mul_body, 0)
        pltpu.sync_copy(rs_v, p_out.at[pl.ds(w * EPWA + j * CC, CC)])
        return carry

    lax.fori_loop(0, GA, body, 0)


def _tc1_body(f_ref, w_ref, d0_ref, d1_ref, m1p_ref, dinv_ref):
    deg = d0_ref[:, :1] + d1_ref[:, :1] + 1.0
    dinv = lax.rsqrt(deg)
    m1 = jnp.dot(f_ref[...], w_ref[...], preferred_element_type=jnp.float32)
    m1p_ref[...] = dinv * m1
    dinv_ref[...] = jnp.broadcast_to(dinv, (BR, 8))


def _tc1(features, W1, deg0, deg1):
    return pl.pallas_call(
        _tc1_body,
        grid=(N // BR,),
        in_specs=[
            pl.BlockSpec((BR, D), lambda i: (i, 0)),
            pl.BlockSpec((D, H), lambda i: (0, 0)),
            pl.BlockSpec((BR, 8), lambda i: (i, 0)),
            pl.BlockSpec((BR, 8), lambda i: (i, 0)),
        ],
        out_specs=[
            pl.BlockSpec((BR, H), lambda i: (i, 0)),
            pl.BlockSpec((BR, 8), lambda i: (i, 0)),
        ],
        out_shape=[
            jax.ShapeDtypeStruct((N, H), jnp.float32),
            jax.ShapeDtypeStruct((N, 8), jnp.float32),
        ],
    )(features, W1, deg0, deg1)


def _tc2_body(a0_ref, a1_ref, m1p_ref, dinv_ref, b1_ref, w2_ref, m2p_ref):
    dinv = dinv_ref[:, :1]
    pre = dinv * (a0_ref[...] + a1_ref[...] + m1p_ref[...]) + b1_ref[...]
    h = jnp.maximum(pre, 0.0)
    m2 = jnp.dot(h, w2_ref[...], preferred_element_type=jnp.float32)
    m2p_ref[...] = dinv * m2


def _tc2(acc0, acc1, m1p, dinv, b1, W2):
    return pl.pallas_call(
        _tc2_body,
        grid=(N // BR,),
        in_specs=[
            pl.BlockSpec((BR, H), lambda i: (i, 0)),
            pl.BlockSpec((BR, H), lambda i: (i, 0)),
            pl.BlockSpec((BR, H), lambda i: (i, 0)),
            pl.BlockSpec((BR, 8), lambda i: (i, 0)),
            pl.BlockSpec((1, H), lambda i: (0, 0)),
            pl.BlockSpec((H, H), lambda i: (0, 0)),
        ],
        out_specs=pl.BlockSpec((BR, H), lambda i: (i, 0)),
        out_shape=jax.ShapeDtypeStruct((N, H), jnp.float32),
    )(acc0, acc1, m1p, dinv, b1, W2)


def _tc3_body(a0_ref, a1_ref, m2p_ref, dinv_ref, b2_ref, reps_ref):
    dinv = dinv_ref[:, :1]
    reps_ref[...] = dinv * (a0_ref[...] + a1_ref[...] + m2p_ref[...]) + b2_ref[...]


def _tc3(acc0, acc1, m2p, dinv, b2):
    return pl.pallas_call(
        _tc3_body,
        grid=(N // BR,),
        in_specs=[
            pl.BlockSpec((BR, H), lambda i: (i, 0)),
            pl.BlockSpec((BR, H), lambda i: (i, 0)),
            pl.BlockSpec((BR, H), lambda i: (i, 0)),
            pl.BlockSpec((BR, 8), lambda i: (i, 0)),
            pl.BlockSpec((1, H), lambda i: (0, 0)),
        ],
        out_specs=pl.BlockSpec((BR, H), lambda i: (i, 0)),
        out_shape=jax.ShapeDtypeStruct((N, H), jnp.float32),
    )(acc0, acc1, m2p, dinv, b2)


def _score_body(p_ref, src_ref, dst_ref, t_ref, num_ref, den_ref, acc_ref):
    i = pl.program_id(0)

    @pl.when(i == 0)
    def _():
        acc_ref[0] = 0.0
        acc_ref[1] = 0.0

    p = jnp.sum(p_ref[...], axis=1, keepdims=True)
    m = src_ref[...] < dst_ref[...]
    acc_ref[0] += jnp.sum(jnp.where(m, (p - t_ref[...]) ** 2, 0.0))
    acc_ref[1] += jnp.sum(m.astype(jnp.float32))

    @pl.when(i == NBK - 1)
    def _():
        num_ref[0, 0] = acc_ref[0]
        den_ref[0, 0] = acc_ref[1]


def _tc_score(P, srcA, dstA, tgt):
    return pl.pallas_call(
        _score_body,
        grid=(NBK,),
        in_specs=[
            pl.BlockSpec((BK, H), lambda i: (i, 0)),
            pl.BlockSpec((BK, 1), lambda i: (i, 0)),
            pl.BlockSpec((BK, 1), lambda i: (i, 0)),
            pl.BlockSpec((BK, 1), lambda i: (i, 0)),
        ],
        out_specs=[
            pl.BlockSpec(memory_space=pltpu.SMEM),
            pl.BlockSpec(memory_space=pltpu.SMEM),
        ],
        out_shape=[
            jax.ShapeDtypeStruct((1, 1), jnp.float32),
            jax.ShapeDtypeStruct((1, 1), jnp.float32),
        ],
        scratch_shapes=[pltpu.SMEM((2,), jnp.float32)],
    )(P, srcA, dstA, tgt)


def _grouped(x, n_real, g, fill):
    pad = NW * g * CC - n_real
    xp = jnp.concatenate([x, jnp.full((pad,), fill, x.dtype)])
    return xp.reshape(NW, g, CC)


def kernel(features, edge_index, neg_edges, W1, b1, W2, b2):
    src = edge_index[0]
    dst = edge_index[1]

    src_p = _grouped(src, E, G, 0)
    dst_p = _grouped(dst, E, G, DUMMY_ROW)

    srcA = jnp.concatenate([src, neg_edges[0]])
    dstA = jnp.concatenate([dst, neg_edges[1]])
    tgtA = jnp.concatenate([jnp.ones((E,), jnp.float32),
                            jnp.zeros((NEG,), jnp.float32)])
    srcA_p = _grouped(srcA, EA, GA, 0)
    dstA_p = _grouped(dstA, EA, GA, 0)
    tgt_p = _grouped(tgtA, EA, GA, 0.0)

    zeros8 = jnp.zeros((N_PAD, 8), jnp.float32)
    zerosH = jnp.zeros((N_PAD, H), jnp.float32)
    ones8 = jnp.ones((CC, 8), jnp.float32)

    deg_pair = _deg_kernel(dst_p, zeros8, ones8)
    deg0 = deg_pair[0, :N, :]
    deg1 = deg_pair[1, :N, :]

    m1p, dinv = _tc1(features, W1, deg0, deg1)

    acc1 = _prop_kernel(m1p, src_p, dst_p, zerosH)

    m2p = _tc2(acc1[0, :N, :], acc1[1, :N, :], m1p, dinv, b1.reshape(1, H), W2)

    acc2 = _prop_kernel(m2p, src_p, dst_p, zerosH)

    reps = _tc3(acc2[0, :N, :], acc2[1, :N, :], m2p, dinv, b2.reshape(1, H))

    P = _edgeprod_kernel(reps, srcA_p, dstA_p)

    num, den = _tc_score(P, srcA_p.reshape(EA_PAD, 1), dstA_p.reshape(EA_PAD, 1),
                         tgt_p.reshape(EA_PAD, 1))
    rec_loss = (num[0, 0] * jnp.float32(N)) / den[0, 0]
    return reps, rec_loss


# speedup vs baseline: 2.4729x; 1.2654x over previous
import functools

import jax
import jax.numpy as jnp
from jax import lax
from jax.experimental import pallas as pl
from jax.experimental.pallas import tpu as pltpu
from jax.experimental.pallas import tpu_sc as plsc

N = 10000
D = 128
H = 64
E = 320000
NEG = 50000

NC = 2
NS = 16
NW = NC * NS

CC = 512
N_PAD = 10240
RPS = N_PAD // NS
DUMMY_ROW = N

G = -(-E // (NW * CC))
E_PAD = G * NW * CC
EPW = G * CC

EA = E + NEG
GA = -(-EA // (NW * CC))
EA_PAD = GA * NW * CC
EPWA = GA * CC

BR = 1000
BK = 2048
NBK = EA_PAD // BK


def _mesh():
    return plsc.VectorSubcoreMesh(
        core_axis_name="c", subcore_axis_name="s", num_cores=NC, num_subcores=NS
    )


_SC_PARAMS = pltpu.CompilerParams(use_tc_tiling_on_sc=False)


@functools.partial(
    pl.kernel,
    out_type=jax.ShapeDtypeStruct((NC, N_PAD, 8), jnp.float32),
    mesh=_mesh(),
    compiler_params=_SC_PARAMS,
    scratch_types=[
        pltpu.VMEM_SHARED((N_PAD, 8), jnp.float32),
        pltpu.VMEM((G, CC), jnp.int32),
        pltpu.VMEM((CC, 8), jnp.float32),
    ],
)
def _deg_kernel(dst_hbm, zeros_hbm, ones_hbm, out_hbm, acc_sh, dst_v, ones_v):
    c = lax.axis_index("c")
    s = lax.axis_index("s")
    w = c * NS + s
    pltpu.sync_copy(zeros_hbm.at[pl.ds(s * RPS, RPS)], acc_sh.at[pl.ds(s * RPS, RPS)])
    pltpu.sync_copy(ones_hbm, ones_v)
    pltpu.sync_copy(dst_hbm.at[w], dst_v)
    plsc.subcore_barrier()

    def body(j, carry):
        pltpu.sync_copy(ones_v, acc_sh.at[dst_v.at[j]], add=True)
        return carry

    lax.fori_loop(0, G, body, 0)
    plsc.subcore_barrier()
    pltpu.sync_copy(acc_sh.at[pl.ds(s * RPS, RPS)], out_hbm.at[c].at[pl.ds(s * RPS, RPS)])


@functools.partial(
    pl.kernel,
    out_type=jax.ShapeDtypeStruct((NC, N_PAD, H), jnp.float32),
    mesh=_mesh(),
    compiler_params=_SC_PARAMS,
    scratch_types=[
        pltpu.VMEM_SHARED((N_PAD, H), jnp.float32),
        pltpu.VMEM((G, CC), jnp.int32),
        pltpu.VMEM((G, CC), jnp.int32),
        pltpu.VMEM((CC, H), jnp.float32),
        pltpu.SemaphoreType.DMA,
    ],
)
def _prop_kernel(m_hbm, src_hbm, dst_hbm, zeros_hbm, out_hbm,
                 acc_sh, src_v, dst_v, rows_v, sem):
    c = lax.axis_index("c")
    s = lax.axis_index("s")
    w = c * NS + s
    pltpu.sync_copy(zeros_hbm.at[pl.ds(s * RPS, RPS)], acc_sh.at[pl.ds(s * RPS, RPS)])
    pltpu.sync_copy(src_hbm.at[w], src_v)
    pltpu.sync_copy(dst_hbm.at[w], dst_v)
    plsc.subcore_barrier()

    def body(j, carry):
        pltpu.async_copy(m_hbm.at[src_v.at[j]], rows_v, sem).wait()
        pltpu.sync_copy(rows_v, acc_sh.at[dst_v.at[j]], add=True)
        return carry

    lax.fori_loop(0, G, body, 0)
    plsc.subcore_barrier()
    pltpu.sync_copy(acc_sh.at[pl.ds(s * RPS, RPS)], out_hbm.at[c].at[pl.ds(s * RPS, RPS)])


@functools.partial(
    pl.kernel,
    out_type=jax.ShapeDtypeStruct((EA_PAD, H), jnp.float32),
    mesh=_mesh(),
    compiler_params=_SC_PARAMS,
    scratch_types=[
        pltpu.VMEM_SHARED((N, H), jnp.float32),
        pltpu.VMEM((GA, CC), jnp.int32),
        pltpu.VMEM((GA, CC), jnp.int32),
        pltpu.VMEM((CC, H), jnp.float32),
        pltpu.VMEM((CC, H), jnp.float32),
        pltpu.SemaphoreType.DMA,
        pltpu.SemaphoreType.DMA,
    ],
)
def _edgeprod_kernel(reps_hbm, src_hbm, dst_hbm, p_out,
                     reps_sh, src_v, dst_v, rs_v, rd_v, sem_s, sem_d):
    c = lax.axis_index("c")
    s = lax.axis_index("s")
    w = c * NS + s
    pltpu.sync_copy(reps_hbm.at[pl.ds(s * (N // NS), N // NS)],
                    reps_sh.at[pl.ds(s * (N // NS), N // NS)])
    pltpu.sync_copy(src_hbm.at[w], src_v)
    pltpu.sync_copy(dst_hbm.at[w], dst_v)
    plsc.subcore_barrier()

    def body(j, carry):
        cs = pltpu.async_copy(reps_sh.at[src_v.at[j]], rs_v, sem_s)
        cd = pltpu.async_copy(reps_sh.at[dst_v.at[j]], rd_v, sem_d)
        cs.wait()
        cd.wait()

        def mul_body(i0, carry2):
            for ee in range(4):
                for k in range(H // 16):
                    e = i0 * 4 + ee
                    sl = pl.ds(k * 16, 16)
                    rs_v[e, sl] = rs_v[e, sl] * rd_v[e, sl]
            return carry2

        lax.fori_loop(0, CC // 4, mul_body, 0)
        pltpu.sync_copy(rs_v, p_out.at[pl.ds(w * EPWA + j * CC, CC)])
        return carry

    lax.fori_loop(0, GA, body, 0)


def _tc1_body(f_ref, w_ref, d0_ref, d1_ref, m1p_ref, dinv_ref):
    deg = d0_ref[:, :1] + d1_ref[:, :1] + 1.0
    dinv = lax.rsqrt(deg)
    m1 = jnp.dot(f_ref[...], w_ref[...], preferred_element_type=jnp.float32)
    m1p_ref[...] = dinv * m1
    dinv_ref[...] = jnp.broadcast_to(dinv, (BR, 8))


def _tc1(features, W1, deg0, deg1):
    return pl.pallas_call(
        _tc1_body,
        grid=(N // BR,),
        in_specs=[
            pl.BlockSpec((BR, D), lambda i: (i, 0)),
            pl.BlockSpec((D, H), lambda i: (0, 0)),
            pl.BlockSpec((BR, 8), lambda i: (i, 0)),
            pl.BlockSpec((BR, 8), lambda i: (i, 0)),
        ],
        out_specs=[
            pl.BlockSpec((BR, H), lambda i: (i, 0)),
            pl.BlockSpec((BR, 8), lambda i: (i, 0)),
        ],
        out_shape=[
            jax.ShapeDtypeStruct((N, H), jnp.float32),
            jax.ShapeDtypeStruct((N, 8), jnp.float32),
        ],
    )(features, W1, deg0, deg1)


def _tc2_body(a0_ref, a1_ref, m1p_ref, dinv_ref, b1_ref, w2_ref, m2p_ref):
    dinv = dinv_ref[:, :1]
    pre = dinv * (a0_ref[...] + a1_ref[...] + m1p_ref[...]) + b1_ref[...]
    h = jnp.maximum(pre, 0.0)
    m2 = jnp.dot(h, w2_ref[...], preferred_element_type=jnp.float32)
    m2p_ref[...] = dinv * m2


def _tc2(acc0, acc1, m1p, dinv, b1, W2):
    return pl.pallas_call(
        _tc2_body,
        grid=(N // BR,),
        in_specs=[
            pl.BlockSpec((BR, H), lambda i: (i, 0)),
            pl.BlockSpec((BR, H), lambda i: (i, 0)),
            pl.BlockSpec((BR, H), lambda i: (i, 0)),
            pl.BlockSpec((BR, 8), lambda i: (i, 0)),
            pl.BlockSpec((1, H), lambda i: (0, 0)),
            pl.BlockSpec((H, H), lambda i: (0, 0)),
        ],
        out_specs=pl.BlockSpec((BR, H), lambda i: (i, 0)),
        out_shape=jax.ShapeDtypeStruct((N, H), jnp.float32),
    )(acc0, acc1, m1p, dinv, b1, W2)


def _tc3_body(a0_ref, a1_ref, m2p_ref, dinv_ref, b2_ref, reps_ref):
    dinv = dinv_ref[:, :1]
    reps_ref[...] = dinv * (a0_ref[...] + a1_ref[...] + m2p_ref[...]) + b2_ref[...]


def _tc3(acc0, acc1, m2p, dinv, b2):
    return pl.pallas_call(
        _tc3_body,
        grid=(N // BR,),
        in_specs=[
            pl.BlockSpec((BR, H), lambda i: (i, 0)),
            pl.BlockSpec((BR, H), lambda i: (i, 0)),
            pl.BlockSpec((BR, H), lambda i: (i, 0)),
            pl.BlockSpec((BR, 8), lambda i: (i, 0)),
            pl.BlockSpec((1, H), lambda i: (0, 0)),
        ],
        out_specs=pl.BlockSpec((BR, H), lambda i: (i, 0)),
        out_shape=jax.ShapeDtypeStruct((N, H), jnp.float32),
    )(acc0, acc1, m2p, dinv, b2)


def _score_body(p_ref, src_ref, dst_ref, t_ref, num_ref, den_ref, acc_ref):
    i = pl.program_id(0)

    @pl.when(i == 0)
    def _():
        acc_ref[0] = 0.0
        acc_ref[1] = 0.0

    p = jnp.sum(p_ref[...], axis=1, keepdims=True)
    m = src_ref[...] < dst_ref[...]
    acc_ref[0] += jnp.sum(jnp.where(m, (p - t_ref[...]) ** 2, 0.0))
    acc_ref[1] += jnp.sum(m.astype(jnp.float32))

    @pl.when(i == NBK - 1)
    def _():
        num_ref[0, 0] = acc_ref[0]
        den_ref[0, 0] = acc_ref[1]


def _tc_score(P, srcA, dstA, tgt):
    return pl.pallas_call(
        _score_body,
        grid=(NBK,),
        in_specs=[
            pl.BlockSpec((BK, H), lambda i: (i, 0)),
            pl.BlockSpec((BK, 1), lambda i: (i, 0)),
            pl.BlockSpec((BK, 1), lambda i: (i, 0)),
            pl.BlockSpec((BK, 1), lambda i: (i, 0)),
        ],
        out_specs=[
            pl.BlockSpec(memory_space=pltpu.SMEM),
            pl.BlockSpec(memory_space=pltpu.SMEM),
        ],
        out_shape=[
            jax.ShapeDtypeStruct((1, 1), jnp.float32),
            jax.ShapeDtypeStruct((1, 1), jnp.float32),
        ],
        scratch_shapes=[pltpu.SMEM((2,), jnp.float32)],
    )(P, srcA, dstA, tgt)


def _grouped(x, n_real, g, fill):
    pad = NW * g * CC - n_real
    xp = jnp.concatenate([x, jnp.full((pad,), fill, x.dtype)])
    return xp.reshape(NW, g, CC)


def kernel(features, edge_index, neg_edges, W1, b1, W2, b2):
    src = edge_index[0]
    dst = edge_index[1]

    src_p = _grouped(src, E, G, 0)
    dst_p = _grouped(dst, E, G, DUMMY_ROW)

    srcA = jnp.concatenate([src, neg_edges[0]])
    dstA = jnp.concatenate([dst, neg_edges[1]])
    tgtA = jnp.concatenate([jnp.ones((E,), jnp.float32),
                            jnp.zeros((NEG,), jnp.float32)])
    srcA_p = _grouped(srcA, EA, GA, 0)
    dstA_p = _grouped(dstA, EA, GA, 0)
    tgt_p = _grouped(tgtA, EA, GA, 0.0)

    zeros8 = jnp.zeros((N_PAD, 8), jnp.float32)
    zerosH = jnp.zeros((N_PAD, H), jnp.float32)
    ones8 = jnp.ones((CC, 8), jnp.float32)

    deg_pair = _deg_kernel(dst_p, zeros8, ones8)
    deg0 = deg_pair[0, :N, :]
    deg1 = deg_pair[1, :N, :]

    m1p, dinv = _tc1(features, W1, deg0, deg1)

    acc1 = _prop_kernel(m1p, src_p, dst_p, zerosH)

    m2p = _tc2(acc1[0, :N, :], acc1[1, :N, :], m1p, dinv, b1.reshape(1, H), W2)

    acc2 = _prop_kernel(m2p, src_p, dst_p, zerosH)

    reps = _tc3(acc2[0, :N, :], acc2[1, :N, :], m2p, dinv, b2.reshape(1, H))

    P = _edgeprod_kernel(reps, srcA_p, dstA_p)

    num, den = _tc_score(P, srcA_p.reshape(EA_PAD, 1), dstA_p.reshape(EA_PAD, 1),
                         tgt_p.reshape(EA_PAD, 1))
    rec_loss = (num[0, 0] * jnp.float32(N)) / den[0, 0]
    return reps, rec_loss
